# baseline (speedup 1.0000x reference)
import functools

import jax
import jax.numpy as jnp
from jax import lax
from jax.experimental import pallas as pl
from jax.experimental.pallas import tpu as pltpu
from jax.experimental.pallas import tpu_sc as plsc

_OFFSET = 16384

_NC = 2
_NS = 16
_NW = _NC * _NS
_LANES = 16

_CH = 16384


@functools.lru_cache(maxsize=None)
def _make_offset_kernel(total: int, dtype_name: str):
    dtype = jnp.dtype(dtype_name)
    n_full = total // _CH
    rem_start = n_full * _CH
    rem16 = ((total - rem_start) // _LANES) * _LANES
    tail = total - rem_start - rem16
    tail_start = rem_start + rem16
    max_rounds = -(-n_full // _NW)

    mesh = plsc.VectorSubcoreMesh(core_axis_name="c", subcore_axis_name="s",
                                  num_cores=_NC, num_subcores=_NS)

    scratch = [pltpu.VMEM((_CH,), dtype)]
    if rem16:
        scratch.append(pltpu.VMEM((rem16,), dtype))
    if tail:
        scratch.append(pltpu.VMEM((_LANES,), dtype))

    @functools.partial(
        pl.kernel,
        out_type=jax.ShapeDtypeStruct((total,), dtype),
        mesh=mesh,
        scratch_types=scratch,
    )
    def offset_kernel(x_hbm, out_hbm, *bufs):
        buf = bufs[0]
        wid = lax.axis_index("s") * _NC + lax.axis_index("c")

        def add_slices(ref, nvec):
            def body(j, _):
                sl = pl.ds(j * _LANES, _LANES)
                ref[sl] = ref[sl] + dtype.type(_OFFSET)
                return _
            lax.fori_loop(0, nvec, body, None)

        for i in range(max_rounds):
            cid = wid + i * _NW
            @pl.when(cid < n_full)
            def _():
                base = cid * _CH
                pltpu.sync_copy(x_hbm.at[pl.ds(base, _CH)], buf)
                add_slices(buf, _CH // _LANES)
                pltpu.sync_copy(buf, out_hbm.at[pl.ds(base, _CH)])

        if rem16:
            rbuf = bufs[1]
            @pl.when(wid == 0)
            def _():
                pltpu.sync_copy(x_hbm.at[pl.ds(rem_start, rem16)], rbuf)
                add_slices(rbuf, rem16 // _LANES)
                pltpu.sync_copy(rbuf, out_hbm.at[pl.ds(rem_start, rem16)])

        if tail:
            tbuf = bufs[-1]
            @pl.when(wid == 1)
            def _():
                pltpu.sync_copy(x_hbm.at[pl.ds(tail_start, tail)],
                                tbuf.at[pl.ds(0, tail)])
                tbuf[...] = tbuf[...] + dtype.type(_OFFSET)
                pltpu.sync_copy(tbuf.at[pl.ds(0, tail)],
                                out_hbm.at[pl.ds(tail_start, tail)])

    return offset_kernel


def kernel(indices, values):
    total = indices.shape[0] * indices.shape[1]
    flat = indices.reshape(total)
    k = _make_offset_kernel(total, str(flat.dtype))
    new_flat = k(flat)
    return (new_flat.reshape(indices.shape), values)


# speedup vs baseline: 1.0084x; 1.0084x over previous
import functools

import jax
import jax.numpy as jnp
from jax.experimental import pallas as pl
from jax.experimental.pallas import tpu as pltpu

_OFFSET = 16384


def _body(x_ref, v_ref, ox_ref, ov_ref):
    ox_ref[...] = x_ref[...] + x_ref.dtype.type(_OFFSET)
    ov_ref[...] = v_ref[...]


@functools.lru_cache(maxsize=None)
def _make_call(total: int, nnz: int, idx_dtype: str, val_dtype: str,
               n_blocks: int):
    bx = -(-total // n_blocks)
    bx = -(-bx // 1024) * 1024
    bv = -(-nnz // n_blocks)
    bv = -(-bv // 1024) * 1024
    grid = max(-(-total // bx), -(-nnz // bv))
    return pl.pallas_call(
        _body,
        grid=(grid,),
        in_specs=[
            pl.BlockSpec((bx,), lambda i: (i,)),
            pl.BlockSpec((bv,), lambda i: (i,)),
        ],
        out_specs=[
            pl.BlockSpec((bx,), lambda i: (i,)),
            pl.BlockSpec((bv,), lambda i: (i,)),
        ],
        out_shape=[
            jax.ShapeDtypeStruct((total,), jnp.dtype(idx_dtype)),
            jax.ShapeDtypeStruct((nnz,), jnp.dtype(val_dtype)),
        ],
        compiler_params=pltpu.CompilerParams(
            dimension_semantics=("arbitrary",),
        ),
    )


def kernel(indices, values):
    nnz = indices.shape[0]
    total = nnz * indices.shape[1]
    flat = indices.reshape(total)
    call = _make_call(total, nnz, str(indices.dtype), str(values.dtype), 16)
    new_flat, new_values = call(flat, values)
    return (new_flat.reshape(indices.shape), new_values)


# speedup vs baseline: 2.1062x; 2.0888x over previous
import functools

import jax
import jax.numpy as jnp
from jax.experimental import pallas as pl
from jax.experimental.pallas import tpu as pltpu

_OFFSET = 16384


def _body(x_ref, ox_ref):
    ox_ref[...] = x_ref[...] + x_ref.dtype.type(_OFFSET)


@functools.lru_cache(maxsize=None)
def _make_call(nnz: int, ncols: int, idx_dtype: str, br: int):
    idt = jnp.dtype(idx_dtype)
    grid = -(-nnz // br)
    return pl.pallas_call(
        _body,
        grid=(grid,),
        in_specs=[pl.BlockSpec((br, ncols), lambda i: (i, 0))],
        out_specs=pl.BlockSpec((br, ncols), lambda i: (i, 0)),
        out_shape=jax.ShapeDtypeStruct((nnz, ncols), idt),
        compiler_params=pltpu.CompilerParams(
            dimension_semantics=("arbitrary",),
        ),
    )


def kernel(indices, values):
    nnz, ncols = indices.shape
    call = _make_call(nnz, ncols, str(indices.dtype), 2048)
    return (call(indices), values)
